# Initial kernel scaffold; baseline (speedup 1.0000x reference)
import jax
import jax.numpy as jnp
from jax.experimental import pallas as pl


def kernel(shift, features, type_mask, edge_metapath_indices, w_weight, w_bias):
    raise NotImplementedError("write your pallas kernel here")


# speedup vs baseline: 14.5287x; 14.5287x over previous
import functools

import jax
import jax.numpy as jnp
from jax import lax
from jax.experimental import pallas as pl
from jax.experimental.pallas import tpu as pltpu
from jax.experimental.pallas import tpu_sc as plsc

NUM_HEADS = 8
OUT_DIM = 256
NH_DIM = NUM_HEADS * OUT_DIM
N_NODES = 50000
N_PATHS = 50000
PATH_LEN = 4
BS = 2048

NW = 32
SEG_PER_W = BS // NW
CHUNK = 32
ROWS = CHUNK * PATH_LEN
DUMMY = SEG_PER_W
NC16 = OUT_DIM // 16


def _permute(vec, idx):
    return vec.at[idx].get(mode="promise_in_bounds")


def _sc_segment_sum(shift_pad, emi_flat, features, interpret=False):
    mesh = plsc.VectorSubcoreMesh(core_axis_name="c", subcore_axis_name="s",
                                  num_cores=2, num_subcores=16)

    @functools.partial(
        pl.kernel,
        out_type=jax.ShapeDtypeStruct((BS, OUT_DIM), jnp.float32),
        mesh=mesh,
        interpret=interpret,
        scratch_types=[
            pltpu.VMEM((SEG_PER_W + 16,), jnp.int32),
            pltpu.VMEM((ROWS,), jnp.int32),
            pltpu.VMEM((ROWS, OUT_DIM), jnp.float32),
            pltpu.VMEM((SEG_PER_W + 1, OUT_DIM), jnp.float32),
            pltpu.SemaphoreType.DMA,
        ],
    )
    def k(shift_hbm, emi_hbm, feat_hbm, out_hbm,
          shift_v, gidx_v, rows_v, acc_v, sem):
        wid = lax.axis_index("s") * 2 + lax.axis_index("c")
        b0 = wid * SEG_PER_W

        pltpu.sync_copy(shift_hbm.at[pl.ds(b0, SEG_PER_W + 16)], shift_v)
        sv = [shift_v[pl.ds(g * 16, 16)] for g in range(SEG_PER_W // 16 + 1)]
        s = sv[0][0]
        e = sv[SEG_PER_W // 16][0]
        lane = [jnp.full((16,), j, jnp.int32) for j in range(16)]
        bounds = [_permute(sv[b // 16], lane[b % 16])
                  for b in range(1, SEG_PER_W + 1)]

        zero16 = jnp.zeros((16,), jnp.float32)
        for r in range(SEG_PER_W + 1):
            for c in range(NC16):
                acc_v[r, pl.ds(c * 16, 16)] = zero16

        a = (s >> 1) << 1
        n_chunks = (e - a + CHUNK - 1) >> 5
        iota = lax.iota(jnp.int32, 16)

        def chunk_body(ci, _):
            base = a + ci * CHUNK
            off = pl.multiple_of(base * PATH_LEN, 8)
            pltpu.sync_copy(emi_hbm.at[pl.ds(off, ROWS)], gidx_v)
            pltpu.async_copy(feat_hbm.at[gidx_v], rows_v, sem).wait()
            for vi in range(CHUNK // 16):
                v = base + vi * 16 + iota
                cnt = jnp.zeros((16,), jnp.int32)
                for sb in bounds:
                    cnt = cnt + jnp.where(v >= sb, 1, 0)
                valid = (v >= s) & (v < e)
                seg = jnp.where(valid, cnt, DUMMY)
                for ln in range(16):
                    sj = seg[ln]
                    for l in range(PATH_LEN):
                        r = vi * 64 + ln * PATH_LEN + l
                        for c in range(NC16):
                            plsc.addupdate(
                                acc_v.at[sj, pl.ds(c * 16, 16)],
                                rows_v[r, pl.ds(c * 16, 16)])
            return 0

        lax.fori_loop(0, n_chunks, chunk_body, 0)

        pltpu.sync_copy(acc_v.at[pl.ds(0, SEG_PER_W)],
                        out_hbm.at[pl.ds(b0, SEG_PER_W)])

    return k(shift_pad, emi_flat, features)


def _tc_body(acc_ref, w_ref, s0_ref, s1_ref, bias_ref, out_ref):
    acc = acc_ref[...] * 0.25
    x = lax.dot_general(acc, w_ref[...], (((1,), (1,)), ((), ())),
                        preferred_element_type=jnp.float32)
    cnt = (s1_ref[...] - s0_ref[...]).astype(jnp.float32)
    x = x + cnt * bias_ref[...]
    out_ref[...] = jnp.where(x > 0, x, jnp.exp(x) - 1.0)


def _tc_head(acc, w_weight, s0, s1, bias):
    bm = 256
    grid = (BS // bm,)
    return pl.pallas_call(
        _tc_body,
        grid=grid,
        in_specs=[
            pl.BlockSpec((bm, OUT_DIM), lambda i: (i, 0)),
            pl.BlockSpec((NH_DIM, OUT_DIM), lambda i: (0, 0)),
            pl.BlockSpec((bm, 1), lambda i: (i, 0)),
            pl.BlockSpec((bm, 1), lambda i: (i, 0)),
            pl.BlockSpec((1, NH_DIM), lambda i: (0, 0)),
        ],
        out_specs=pl.BlockSpec((bm, NH_DIM), lambda i: (i, 0)),
        out_shape=jax.ShapeDtypeStruct((BS, NH_DIM), jnp.float32),
    )(acc, w_weight, s0, s1, bias)


def kernel(shift, features, type_mask, edge_metapath_indices, w_weight, w_bias):
    del type_mask
    shift = shift.astype(jnp.int32)
    shift_pad = jnp.concatenate(
        [shift, jnp.zeros((15,), jnp.int32)])
    emi_flat = jnp.concatenate(
        [edge_metapath_indices.astype(jnp.int32).reshape(-1),
         jnp.zeros((2 * CHUNK * PATH_LEN,), jnp.int32)])

    acc = _sc_segment_sum(shift_pad, emi_flat, features)

    s0 = shift[:-1].reshape(BS, 1)
    s1 = shift[1:].reshape(BS, 1)
    out = _tc_head(acc, w_weight, s0, s1, w_bias.reshape(1, NH_DIM))
    return out.reshape(BS, NUM_HEADS, OUT_DIM)


# speedup vs baseline: 18.1151x; 1.2468x over previous
import functools

import jax
import jax.numpy as jnp
from jax import lax
from jax.experimental import pallas as pl
from jax.experimental.pallas import tpu as pltpu
from jax.experimental.pallas import tpu_sc as plsc

NUM_HEADS = 8
OUT_DIM = 256
NH_DIM = NUM_HEADS * OUT_DIM
N_NODES = 50000
N_PATHS = 50000
PATH_LEN = 4
BS = 2048

NW = 32
SEG_PER_W = BS // NW
CHUNK = 32
ROWS = CHUNK * PATH_LEN
DUMMY = SEG_PER_W
NC16 = OUT_DIM // 16


def _permute(vec, idx):
    return vec.at[idx].get(mode="promise_in_bounds")


def _sc_segment_sum(shift_pad, emi_flat, features, interpret=False):
    mesh = plsc.VectorSubcoreMesh(core_axis_name="c", subcore_axis_name="s",
                                  num_cores=2, num_subcores=16)

    @functools.partial(
        pl.kernel,
        out_type=jax.ShapeDtypeStruct((BS, OUT_DIM), jnp.float32),
        mesh=mesh,
        interpret=interpret,
        scratch_types=[
            pltpu.VMEM((SEG_PER_W + 16,), jnp.int32),
            pltpu.VMEM((ROWS,), jnp.int32),
            pltpu.VMEM((ROWS, OUT_DIM), jnp.float32),
            pltpu.VMEM((SEG_PER_W + 1, OUT_DIM), jnp.float32),
            pltpu.SemaphoreType.DMA,
        ],
    )
    def k(shift_hbm, emi_hbm, feat_hbm, out_hbm,
          shift_v, gidx_v, rows_v, acc_v, sem):
        wid = lax.axis_index("s") * 2 + lax.axis_index("c")
        b0 = wid * SEG_PER_W

        pltpu.sync_copy(shift_hbm.at[pl.ds(b0, SEG_PER_W + 16)], shift_v)
        sv = [shift_v[pl.ds(g * 16, 16)] for g in range(SEG_PER_W // 16 + 1)]
        s = sv[0][0]
        e = sv[SEG_PER_W // 16][0]
        lane = [jnp.full((16,), j, jnp.int32) for j in range(16)]
        bounds = [_permute(sv[b // 16], lane[b % 16])
                  for b in range(1, SEG_PER_W + 1)]

        zero16 = jnp.zeros((16,), jnp.float32)
        for r in range(SEG_PER_W + 1):
            for c in range(NC16):
                acc_v[r, pl.ds(c * 16, 16)] = zero16

        a = (s >> 1) << 1
        n_chunks = (e - a + CHUNK - 1) >> 5
        iota = lax.iota(jnp.int32, 16)

        def chunk_body(ci, _):
            base = a + ci * CHUNK
            off = pl.multiple_of(base * PATH_LEN, 8)
            pltpu.sync_copy(emi_hbm.at[pl.ds(off, ROWS)], gidx_v)
            pltpu.async_copy(feat_hbm.at[gidx_v], rows_v, sem).wait()
            for vi in range(CHUNK // 16):
                v = base + vi * 16 + iota
                cnt = jnp.zeros((16,), jnp.int32)
                for sb in bounds:
                    cnt = cnt + jnp.where(v >= sb, 1, 0)
                valid = (v >= s) & (v < e)
                seg = jnp.where(valid, cnt, DUMMY)
                for ln in range(16):
                    sj = seg[ln]
                    r = vi * 64 + ln * PATH_LEN
                    for c in range(NC16):
                        cs = pl.ds(c * 16, 16)
                        x = ((rows_v[r, cs] + rows_v[r + 1, cs])
                             + (rows_v[r + 2, cs] + rows_v[r + 3, cs]))
                        plsc.addupdate(acc_v.at[sj, cs], x)
            return 0

        lax.fori_loop(0, n_chunks, chunk_body, 0)

        pltpu.sync_copy(acc_v.at[pl.ds(0, SEG_PER_W)],
                        out_hbm.at[pl.ds(b0, SEG_PER_W)])

    return k(shift_pad, emi_flat, features)


def _tc_body(acc_ref, w_ref, s0_ref, s1_ref, bias_ref, out_ref):
    acc = acc_ref[...] * 0.25
    x = lax.dot_general(acc, w_ref[...], (((1,), (1,)), ((), ())),
                        preferred_element_type=jnp.float32)
    cnt = (s1_ref[...] - s0_ref[...]).astype(jnp.float32)
    x = x + cnt * bias_ref[...]
    out_ref[...] = jnp.where(x > 0, x, jnp.exp(x) - 1.0)


def _tc_head(acc, w_weight, s0, s1, bias):
    bm = 256
    grid = (BS // bm,)
    return pl.pallas_call(
        _tc_body,
        grid=grid,
        in_specs=[
            pl.BlockSpec((bm, OUT_DIM), lambda i: (i, 0)),
            pl.BlockSpec((NH_DIM, OUT_DIM), lambda i: (0, 0)),
            pl.BlockSpec((bm, 1), lambda i: (i, 0)),
            pl.BlockSpec((bm, 1), lambda i: (i, 0)),
            pl.BlockSpec((1, NH_DIM), lambda i: (0, 0)),
        ],
        out_specs=pl.BlockSpec((bm, NH_DIM), lambda i: (i, 0)),
        out_shape=jax.ShapeDtypeStruct((BS, NH_DIM), jnp.float32),
    )(acc, w_weight, s0, s1, bias)


def kernel(shift, features, type_mask, edge_metapath_indices, w_weight, w_bias):
    del type_mask
    shift = shift.astype(jnp.int32)
    shift_pad = jnp.concatenate(
        [shift, jnp.zeros((15,), jnp.int32)])
    emi_flat = jnp.concatenate(
        [edge_metapath_indices.astype(jnp.int32).reshape(-1),
         jnp.zeros((2 * CHUNK * PATH_LEN,), jnp.int32)])

    acc = _sc_segment_sum(shift_pad, emi_flat, features)

    s0 = shift[:-1].reshape(BS, 1)
    s1 = shift[1:].reshape(BS, 1)
    out = _tc_head(acc, w_weight, s0, s1, w_bias.reshape(1, NH_DIM))
    return out.reshape(BS, NUM_HEADS, OUT_DIM)


# speedup vs baseline: 20.8294x; 1.1498x over previous
import functools

import jax
import jax.numpy as jnp
from jax import lax
from jax.experimental import pallas as pl
from jax.experimental.pallas import tpu as pltpu
from jax.experimental.pallas import tpu_sc as plsc

NUM_HEADS = 8
OUT_DIM = 256
NH_DIM = NUM_HEADS * OUT_DIM
N_NODES = 50000
N_PATHS = 50000
PATH_LEN = 4
BS = 2048

NW = 32
SEG_PER_W = BS // NW
CHUNK = 16
ROWS = CHUNK * PATH_LEN
DUMMY = SEG_PER_W
NC16 = OUT_DIM // 16


def _permute(vec, idx):
    return vec.at[idx].get(mode="promise_in_bounds")


def _sc_segment_sum(shift_pad, emi_flat, features, interpret=False):
    mesh = plsc.VectorSubcoreMesh(core_axis_name="c", subcore_axis_name="s",
                                  num_cores=2, num_subcores=16)

    @functools.partial(
        pl.kernel,
        out_type=jax.ShapeDtypeStruct((BS, OUT_DIM), jnp.float32),
        mesh=mesh,
        interpret=interpret,
        scratch_types=[
            pltpu.VMEM((SEG_PER_W + 16,), jnp.int32),
            pltpu.VMEM((ROWS,), jnp.int32),
            pltpu.VMEM((ROWS,), jnp.int32),
            pltpu.VMEM((ROWS, OUT_DIM), jnp.float32),
            pltpu.VMEM((ROWS, OUT_DIM), jnp.float32),
            pltpu.VMEM((SEG_PER_W + 1, OUT_DIM), jnp.float32),
            pltpu.SemaphoreType.DMA,
            pltpu.SemaphoreType.DMA,
        ],
    )
    def k(shift_hbm, emi_hbm, feat_hbm, out_hbm,
          shift_v, gidx0_v, gidx1_v, rows0_v, rows1_v, acc_v, sem0, sem1):
        wid = lax.axis_index("s") * 2 + lax.axis_index("c")
        b0 = wid * SEG_PER_W

        pltpu.sync_copy(shift_hbm.at[pl.ds(b0, SEG_PER_W + 16)], shift_v)
        sv = [shift_v[pl.ds(g * 16, 16)] for g in range(SEG_PER_W // 16 + 1)]
        s = sv[0][0]
        e = sv[SEG_PER_W // 16][0]
        lane = [jnp.full((16,), j, jnp.int32) for j in range(16)]
        bounds = [_permute(sv[b // 16], lane[b % 16])
                  for b in range(1, SEG_PER_W + 1)]

        a = (s >> 1) << 1
        n_chunks = (e - a + CHUNK - 1) >> 4
        iota = lax.iota(jnp.int32, 16)
        bufs = ((gidx0_v, rows0_v, sem0), (gidx1_v, rows1_v, sem1))

        def start_gather(c, gidx_v, rows_v, sem):
            base = a + c * CHUNK
            off = pl.multiple_of(base * PATH_LEN, 8)
            pltpu.sync_copy(emi_hbm.at[pl.ds(off, ROWS)], gidx_v)
            pltpu.async_copy(feat_hbm.at[gidx_v], rows_v, sem)

        def accumulate(c, rows_v):
            base = a + c * CHUNK
            for vi in range(CHUNK // 16):
                v = base + vi * 16 + iota
                cnt = jnp.zeros((16,), jnp.int32)
                for sb in bounds:
                    cnt = cnt + jnp.where(v >= sb, 1, 0)
                valid = (v >= s) & (v < e)
                seg = jnp.where(valid, cnt, DUMMY)
                for ln in range(16):
                    sj = seg[ln]
                    r = vi * 64 + ln * PATH_LEN
                    for c16 in range(NC16):
                        cs = pl.ds(c16 * 16, 16)
                        x = ((rows_v[r, cs] + rows_v[r + 1, cs])
                             + (rows_v[r + 2, cs] + rows_v[r + 3, cs]))
                        plsc.addupdate(acc_v.at[sj, cs], x)

        @pl.when(n_chunks > 0)
        def _prime():
            start_gather(0, *bufs[0])

        zero16 = jnp.zeros((16,), jnp.float32)
        for r in range(SEG_PER_W + 1):
            for c in range(NC16):
                acc_v[r, pl.ds(c * 16, 16)] = zero16

        def pair_body(ci2, _):
            for b in range(2):
                c = ci2 * 2 + b
                gidx_v, rows_v, sem = bufs[b]
                ogidx_v, orows_v, osem = bufs[1 - b]

                @pl.when(c < n_chunks)
                def _step():
                    pltpu.make_async_copy(feat_hbm.at[gidx_v], rows_v,
                                          sem).wait()

                    @pl.when(c + 1 < n_chunks)
                    def _prefetch():
                        start_gather(c + 1, ogidx_v, orows_v, osem)

                    accumulate(c, rows_v)
            return 0

        lax.fori_loop(0, (n_chunks + 1) >> 1, pair_body, 0)

        pltpu.sync_copy(acc_v.at[pl.ds(0, SEG_PER_W)],
                        out_hbm.at[pl.ds(b0, SEG_PER_W)])

    return k(shift_pad, emi_flat, features)


def _tc_body(acc_ref, w_ref, s0_ref, s1_ref, bias_ref, out_ref):
    acc = acc_ref[...] * 0.25
    x = lax.dot_general(acc, w_ref[...], (((1,), (1,)), ((), ())),
                        preferred_element_type=jnp.float32)
    cnt = (s1_ref[...] - s0_ref[...]).astype(jnp.float32)
    x = x + cnt * bias_ref[...]
    out_ref[...] = jnp.where(x > 0, x, jnp.exp(x) - 1.0)


def _tc_head(acc, w_weight, s0, s1, bias):
    bm = 256
    grid = (BS // bm,)
    return pl.pallas_call(
        _tc_body,
        grid=grid,
        in_specs=[
            pl.BlockSpec((bm, OUT_DIM), lambda i: (i, 0)),
            pl.BlockSpec((NH_DIM, OUT_DIM), lambda i: (0, 0)),
            pl.BlockSpec((bm, 1), lambda i: (i, 0)),
            pl.BlockSpec((bm, 1), lambda i: (i, 0)),
            pl.BlockSpec((1, NH_DIM), lambda i: (0, 0)),
        ],
        out_specs=pl.BlockSpec((bm, NH_DIM), lambda i: (i, 0)),
        out_shape=jax.ShapeDtypeStruct((BS, NH_DIM), jnp.float32),
    )(acc, w_weight, s0, s1, bias)


def kernel(shift, features, type_mask, edge_metapath_indices, w_weight, w_bias):
    del type_mask
    shift = shift.astype(jnp.int32)
    shift_pad = jnp.concatenate(
        [shift, jnp.zeros((15,), jnp.int32)])
    emi_flat = jnp.concatenate(
        [edge_metapath_indices.astype(jnp.int32).reshape(-1),
         jnp.zeros((2 * CHUNK * PATH_LEN,), jnp.int32)])

    acc = _sc_segment_sum(shift_pad, emi_flat, features)

    s0 = shift[:-1].reshape(BS, 1)
    s1 = shift[1:].reshape(BS, 1)
    out = _tc_head(acc, w_weight, s0, s1, w_bias.reshape(1, NH_DIM))
    return out.reshape(BS, NUM_HEADS, OUT_DIM)


# speedup vs baseline: 46.0114x; 2.2090x over previous
import functools

import jax
import jax.numpy as jnp
from jax import lax
from jax.experimental import pallas as pl
from jax.experimental.pallas import tpu as pltpu
from jax.experimental.pallas import tpu_sc as plsc

NUM_HEADS = 8
OUT_DIM = 256
NH_DIM = NUM_HEADS * OUT_DIM
N_NODES = 50000
N_PATHS = 50000
PATH_LEN = 4
BS = 2048

NW = 32
SEG_PER_W = BS // NW
CHUNK = 32
ROWS = CHUNK * PATH_LEN
DUMMY = SEG_PER_W
NC16 = OUT_DIM // 16


def _permute(vec, idx):
    return vec.at[idx].get(mode="promise_in_bounds")


def _sc_segment_sum(shift_pad, emi_flat, features, interpret=False):
    mesh = plsc.VectorSubcoreMesh(core_axis_name="c", subcore_axis_name="s",
                                  num_cores=2, num_subcores=16)

    @functools.partial(
        pl.kernel,
        out_type=jax.ShapeDtypeStruct((BS, OUT_DIM), jnp.float32),
        mesh=mesh,
        interpret=interpret,
        scratch_types=[
            pltpu.VMEM((SEG_PER_W + 16,), jnp.int32),
            pltpu.VMEM((ROWS,), jnp.int32),
            pltpu.VMEM((ROWS,), jnp.int32),
            pltpu.VMEM((ROWS, OUT_DIM), jnp.float32),
            pltpu.VMEM((ROWS, OUT_DIM), jnp.float32),
            pltpu.VMEM((SEG_PER_W + 1, OUT_DIM), jnp.float32),
            pltpu.VMEM((CHUNK + 16, ), jnp.int32),
            pltpu.SemaphoreType.DMA,
            pltpu.SemaphoreType.DMA,
        ],
    )
    def k(shift_hbm, emi_hbm, feat_hbm, out_hbm,
          shift_v, gidx0_v, gidx1_v, rows0_v, rows1_v, acc_v, segbuf_v,
          sem0, sem1):
        wid = lax.axis_index("s") * 2 + lax.axis_index("c")
        b0 = wid * SEG_PER_W

        pltpu.sync_copy(shift_hbm.at[pl.ds(b0, SEG_PER_W + 16)], shift_v)
        sv = [shift_v[pl.ds(g * 16, 16)] for g in range(SEG_PER_W // 16 + 1)]
        s = sv[0][0]
        e = sv[SEG_PER_W // 16][0]
        lane = [jnp.full((16,), j, jnp.int32) for j in range(16)]
        bounds = [_permute(sv[b // 16], lane[b % 16])
                  for b in range(1, SEG_PER_W + 1)]

        a = (s >> 1) << 1
        n_chunks = (e - a + CHUNK - 1) >> 5
        iota = lax.iota(jnp.int32, 16)
        bufs = ((gidx0_v, rows0_v, sem0), (gidx1_v, rows1_v, sem1))

        def start_gather(c, gidx_v, rows_v, sem):
            base = a + c * CHUNK
            off = pl.multiple_of(base * PATH_LEN, 8)
            pltpu.sync_copy(emi_hbm.at[pl.ds(off, ROWS)], gidx_v)
            pltpu.async_copy(feat_hbm.at[gidx_v], rows_v, sem)

        def accumulate(c, rows_v):
            base = a + c * CHUNK
            for vi in range(CHUNK // 16):
                v = base + vi * 16 + iota
                cnt = jnp.zeros((16,), jnp.int32)
                for sb in bounds:
                    cnt = cnt + jnp.where(v >= sb, 1, 0)
                valid = (v >= s) & (v < e)
                seg = jnp.where(valid, cnt, DUMMY)
                segbuf_v[pl.ds(vi * 16, 16)] = seg

            @plsc.parallel_loop(0, CHUNK, 1, unroll=4)
            def _acc_edge(ln):
                sj = segbuf_v[pl.ds(ln, 16)][0]
                r = ln * PATH_LEN
                for c16 in range(NC16):
                    cs = pl.ds(c16 * 16, 16)
                    x = ((rows_v[r, cs] + rows_v[r + 1, cs])
                         + (rows_v[r + 2, cs] + rows_v[r + 3, cs]))
                    plsc.addupdate(acc_v.at[sj, cs], x)

        @pl.when(n_chunks > 0)
        def _prime():
            start_gather(0, *bufs[0])

        zero16 = jnp.zeros((16,), jnp.float32)
        for r in range(SEG_PER_W + 1):
            for c in range(NC16):
                acc_v[r, pl.ds(c * 16, 16)] = zero16

        def pair_body(ci2, _):
            for b in range(2):
                c = ci2 * 2 + b
                gidx_v, rows_v, sem = bufs[b]
                ogidx_v, orows_v, osem = bufs[1 - b]

                @pl.when(c < n_chunks)
                def _step():
                    pltpu.make_async_copy(feat_hbm.at[gidx_v], rows_v,
                                          sem).wait()

                    @pl.when(c + 1 < n_chunks)
                    def _prefetch():
                        start_gather(c + 1, ogidx_v, orows_v, osem)

                    accumulate(c, rows_v)
            return 0

        lax.fori_loop(0, (n_chunks + 1) >> 1, pair_body, 0)

        pltpu.sync_copy(acc_v.at[pl.ds(0, SEG_PER_W)],
                        out_hbm.at[pl.ds(b0, SEG_PER_W)])

    return k(shift_pad, emi_flat, features)


def _tc_body(acc_ref, w_ref, s0_ref, s1_ref, bias_ref, out_ref):
    acc = acc_ref[...] * 0.25
    x = lax.dot_general(acc, w_ref[...], (((1,), (1,)), ((), ())),
                        preferred_element_type=jnp.float32)
    cnt = (s1_ref[...] - s0_ref[...]).astype(jnp.float32)
    x = x + cnt * bias_ref[...]
    out_ref[...] = jnp.where(x > 0, x, jnp.exp(x) - 1.0)


def _tc_head(acc, w_weight, s0, s1, bias):
    bm = 256
    grid = (BS // bm,)
    return pl.pallas_call(
        _tc_body,
        grid=grid,
        in_specs=[
            pl.BlockSpec((bm, OUT_DIM), lambda i: (i, 0)),
            pl.BlockSpec((NH_DIM, OUT_DIM), lambda i: (0, 0)),
            pl.BlockSpec((bm, 1), lambda i: (i, 0)),
            pl.BlockSpec((bm, 1), lambda i: (i, 0)),
            pl.BlockSpec((1, NH_DIM), lambda i: (0, 0)),
        ],
        out_specs=pl.BlockSpec((bm, NH_DIM), lambda i: (i, 0)),
        out_shape=jax.ShapeDtypeStruct((BS, NH_DIM), jnp.float32),
    )(acc, w_weight, s0, s1, bias)


def kernel(shift, features, type_mask, edge_metapath_indices, w_weight, w_bias):
    del type_mask
    shift = shift.astype(jnp.int32)
    shift_pad = jnp.concatenate(
        [shift, jnp.zeros((15,), jnp.int32)])
    emi_flat = jnp.concatenate(
        [edge_metapath_indices.astype(jnp.int32).reshape(-1),
         jnp.zeros((2 * CHUNK * PATH_LEN,), jnp.int32)])

    acc = _sc_segment_sum(shift_pad, emi_flat, features)

    s0 = shift[:-1].reshape(BS, 1)
    s1 = shift[1:].reshape(BS, 1)
    out = _tc_head(acc, w_weight, s0, s1, w_bias.reshape(1, NH_DIM))
    return out.reshape(BS, NUM_HEADS, OUT_DIM)


# speedup vs baseline: 52.4206x; 1.1393x over previous
import functools

import jax
import jax.numpy as jnp
from jax import lax
from jax.experimental import pallas as pl
from jax.experimental.pallas import tpu as pltpu
from jax.experimental.pallas import tpu_sc as plsc

NUM_HEADS = 8
OUT_DIM = 256
NH_DIM = NUM_HEADS * OUT_DIM
N_NODES = 50000
N_PATHS = 50000
PATH_LEN = 4
BS = 2048

NW = 32
SEG_PER_W = BS // NW
CHUNK = 32
ROWS = CHUNK * PATH_LEN
DUMMY = SEG_PER_W
NC16 = OUT_DIM // 16


def _permute(vec, idx):
    return vec.at[idx].get(mode="promise_in_bounds")


def _sc_segment_sum(shift_pad, emi_flat, features, interpret=False):
    mesh = plsc.VectorSubcoreMesh(core_axis_name="c", subcore_axis_name="s",
                                  num_cores=2, num_subcores=16)

    @functools.partial(
        pl.kernel,
        out_type=jax.ShapeDtypeStruct((BS, OUT_DIM), jnp.float32),
        mesh=mesh,
        interpret=interpret,
        scratch_types=[
            pltpu.VMEM((SEG_PER_W + 16,), jnp.int32),
            pltpu.VMEM((ROWS,), jnp.int32),
            pltpu.VMEM((ROWS,), jnp.int32),
            pltpu.VMEM((ROWS, OUT_DIM), jnp.float32),
            pltpu.VMEM((ROWS, OUT_DIM), jnp.float32),
            pltpu.VMEM((SEG_PER_W + 1, OUT_DIM), jnp.float32),
            pltpu.VMEM((CHUNK + 16, ), jnp.int32),
            pltpu.SemaphoreType.DMA,
            pltpu.SemaphoreType.DMA,
            pltpu.SemaphoreType.DMA,
            pltpu.SemaphoreType.DMA,
        ],
    )
    def k(shift_hbm, emi_hbm, feat_hbm, out_hbm,
          shift_v, gidx0_v, gidx1_v, rows0_v, rows1_v, acc_v, segbuf_v,
          sem0, sem1, gsem0, gsem1):
        wid = lax.axis_index("s") * 2 + lax.axis_index("c")
        b0 = wid * SEG_PER_W

        pltpu.sync_copy(shift_hbm.at[pl.ds(b0, SEG_PER_W + 16)], shift_v)
        sv = [shift_v[pl.ds(g * 16, 16)] for g in range(SEG_PER_W // 16 + 1)]
        s = sv[0][0]
        e = sv[SEG_PER_W // 16][0]
        lane = [jnp.full((16,), j, jnp.int32) for j in range(16)]
        bounds = [_permute(sv[b // 16], lane[b % 16])
                  for b in range(1, SEG_PER_W + 1)]

        a = (s >> 1) << 1
        n_chunks = (e - a + CHUNK - 1) >> 5
        iota = lax.iota(jnp.int32, 16)
        bufs = ((gidx0_v, rows0_v, sem0, gsem0), (gidx1_v, rows1_v, sem1,
                                                  gsem1))
        last_base = jnp.minimum(a + (n_chunks - 1) * CHUNK,
                                N_PATHS - CHUNK)

        def chunk_base(c):
            ab = a + c * CHUNK
            return jnp.where(c == n_chunks - 1, last_base, ab)

        def start_gidx(c, gidx_v, gsem):
            off = pl.multiple_of(chunk_base(c) * PATH_LEN, 8)
            pltpu.async_copy(emi_hbm.at[pl.ds(off, ROWS)], gidx_v, gsem)

        def start_gather(gidx_v, rows_v, sem, gsem):
            pltpu.make_async_copy(emi_hbm.at[pl.ds(0, ROWS)], gidx_v,
                                  gsem).wait()
            pltpu.async_copy(feat_hbm.at[gidx_v], rows_v, sem)

        def accumulate(c, rows_v):
            base = chunk_base(c)
            lo = jnp.where(c == n_chunks - 1,
                           jnp.maximum(s, a + (n_chunks - 1) * CHUNK), s)
            for vi in range(CHUNK // 16):
                v = base + vi * 16 + iota
                cnt = jnp.zeros((16,), jnp.int32)
                for sb in bounds:
                    cnt = cnt + jnp.where(v >= sb, 1, 0)
                valid = (v >= lo) & (v < e)
                seg = jnp.where(valid, cnt, DUMMY)
                segbuf_v[pl.ds(vi * 16, 16)] = seg

            @plsc.parallel_loop(0, CHUNK, 1, unroll=4)
            def _acc_edge(ln):
                sj = segbuf_v[pl.ds(ln, 16)][0]
                r = ln * PATH_LEN
                for c16 in range(NC16):
                    cs = pl.ds(c16 * 16, 16)
                    x = ((rows_v[r, cs] + rows_v[r + 1, cs])
                         + (rows_v[r + 2, cs] + rows_v[r + 3, cs]))
                    plsc.addupdate(acc_v.at[sj, cs], x)

        @pl.when(n_chunks > 0)
        def _prime():
            start_gidx(0, bufs[0][0], bufs[0][3])
            start_gather(*bufs[0])

            @pl.when(n_chunks > 1)
            def _prime2():
                start_gidx(1, bufs[1][0], bufs[1][3])

        zero16 = jnp.zeros((16,), jnp.float32)
        for r in range(SEG_PER_W + 1):
            for c in range(NC16):
                acc_v[r, pl.ds(c * 16, 16)] = zero16

        def pair_body(ci2, _):
            for b in range(2):
                c = ci2 * 2 + b
                gidx_v, rows_v, sem, gsem = bufs[b]

                @pl.when(c < n_chunks)
                def _step():
                    pltpu.make_async_copy(feat_hbm.at[gidx_v], rows_v,
                                          sem).wait()

                    @pl.when(c + 1 < n_chunks)
                    def _prefetch():
                        start_gather(*bufs[1 - b])

                        @pl.when(c + 2 < n_chunks)
                        def _prefetch_gidx():
                            start_gidx(c + 2, gidx_v, gsem)

                    accumulate(c, rows_v)
            return 0

        lax.fori_loop(0, (n_chunks + 1) >> 1, pair_body, 0)

        pltpu.sync_copy(acc_v.at[pl.ds(0, SEG_PER_W)],
                        out_hbm.at[pl.ds(b0, SEG_PER_W)])

    return k(shift_pad, emi_flat, features)


def _tc_body(acc_ref, w_ref, s0_ref, s1_ref, bias_ref, out_ref):
    acc = acc_ref[...] * 0.25
    x = lax.dot_general(acc, w_ref[...], (((1,), (1,)), ((), ())),
                        preferred_element_type=jnp.float32)
    cnt = (s1_ref[...] - s0_ref[...]).astype(jnp.float32)
    x = x + cnt * bias_ref[...]
    out_ref[...] = jnp.where(x > 0, x, jnp.exp(x) - 1.0)


def _tc_head(acc, w_weight, s0, s1, bias):
    bm = 256
    grid = (BS // bm,)
    return pl.pallas_call(
        _tc_body,
        grid=grid,
        in_specs=[
            pl.BlockSpec((bm, OUT_DIM), lambda i: (i, 0)),
            pl.BlockSpec((NH_DIM, OUT_DIM), lambda i: (0, 0)),
            pl.BlockSpec((bm, 1), lambda i: (i, 0)),
            pl.BlockSpec((bm, 1), lambda i: (i, 0)),
            pl.BlockSpec((1, NH_DIM), lambda i: (0, 0)),
        ],
        out_specs=pl.BlockSpec((bm, NH_DIM), lambda i: (i, 0)),
        out_shape=jax.ShapeDtypeStruct((BS, NH_DIM), jnp.float32),
    )(acc, w_weight, s0, s1, bias)


def kernel(shift, features, type_mask, edge_metapath_indices, w_weight, w_bias):
    del type_mask
    shift = shift.astype(jnp.int32)
    shift_pad = jnp.concatenate(
        [shift, jnp.zeros((15,), jnp.int32)])
    emi_flat = edge_metapath_indices.astype(jnp.int32).reshape(-1)

    acc = _sc_segment_sum(shift_pad, emi_flat, features)

    s0 = shift[:-1].reshape(BS, 1)
    s1 = shift[1:].reshape(BS, 1)
    out = _tc_head(acc, w_weight, s0, s1, w_bias.reshape(1, NH_DIM))
    return out.reshape(BS, NUM_HEADS, OUT_DIM)
